# baseline (speedup 1.0000x reference)
import jax
import jax.numpy as jnp
from jax import lax
from jax.experimental import pallas as pl
from jax.experimental.pallas import tpu as pltpu
from jax.experimental.pallas import tpu_sc as plsc

_VOCAB = 1000000
_CHUNK = 16
_NCH = 4
_ROWS = 8388608 // (_NCH * _CHUNK) // 2
_BATCH = 16384

_INFO = plsc.get_sparse_core_info()
_NC = _INFO.num_cores
_NS = _INFO.num_subcores
_NW = _NC * _NS
_NB = _BATCH // _NW
_NI = _NB * _NCH


def _body(t0_hbm, t1_hbm, r01a_hbm, out_hbm,
          r0_v, r1_v, i0_v, i1_v, p0_v, p1_v, q_v, s0, s1):
    wid = lax.axis_index("s") * _NC + lax.axis_index("c")
    base = wid * _NI

    pltpu.sync_copy(r01a_hbm.at[pl.ds(base, _NI)], r0_v)
    pltpu.sync_copy(r01a_hbm.at[pl.ds(_BATCH * _NCH + base, _NI)], r1_v)

    lane = lax.iota(jnp.int32, 16)
    col = lane & 3

    def repack(i, _):
        i0_v[pl.ds(i * 16, 16)] = r0_v[pl.ds(i * 16, 16)] & (_ROWS - 1)
        i1_v[pl.ds(i * 16, 16)] = r1_v[pl.ds(i * 16, 16)] & (_ROWS - 1)
        return 0

    lax.fori_loop(0, _NI // 16, repack, 0)

    _PS = _NI // 4
    for p in range(4):
        cp0 = pltpu.async_copy(
            t0_hbm.at[i0_v.at[pl.ds(p * _PS, _PS)]], p0_v, s0)
        cp1 = pltpu.async_copy(
            t1_hbm.at[i1_v.at[pl.ds(p * _PS, _PS)]], p1_v, s1)
        cp0.wait()
        cp1.wait()

        def add(u, _):
            for k in range(4):
                j = u * 4 + k
                q_v[p * 128 + u, pl.ds(k * 16, 16)] = (
                    p0_v[j, k] + p1_v[j, k])
            return 0

        lax.fori_loop(0, _PS // 4, add, 0)

    pltpu.sync_copy(q_v, out_hbm.at[pl.ds(wid * _NB, _NB)])


@jax.jit
def _run(x, table0, table1, h0, h1):
    rows0 = jnp.take(h0, x, axis=0)
    rows1 = jnp.take(h1, x, axis=0)
    r01f = jnp.concatenate([rows0, rows1], axis=0).reshape(2 * _BATCH * _NCH)

    kern = pl.kernel(
        _body,
        out_type=jax.ShapeDtypeStruct((_BATCH, _NCH * _CHUNK), jnp.float32),
        mesh=plsc.VectorSubcoreMesh(core_axis_name="c", subcore_axis_name="s"),
        compiler_params=pltpu.CompilerParams(use_tc_tiling_on_sc=False),
        scratch_types=[
            pltpu.VMEM((_NI,), jnp.int32),
            pltpu.VMEM((_NI,), jnp.int32),
            pltpu.VMEM((_NI,), jnp.int32),
            pltpu.VMEM((_NI,), jnp.int32),
            pltpu.VMEM((_NI // 4, _NCH, _CHUNK), jnp.float32),
            pltpu.VMEM((_NI // 4, _NCH, _CHUNK), jnp.float32),
            pltpu.VMEM((_NB, _NCH * _CHUNK), jnp.float32),
            pltpu.SemaphoreType.DMA,
            pltpu.SemaphoreType.DMA,
        ],
    )
    return kern(table0, table1, r01f)


def kernel(x, table0, table1, h0, h1):
    return _run(x, table0, table1, h0, h1)


# speedup vs baseline: 1.2432x; 1.2432x over previous
import jax
import jax.numpy as jnp
from jax import lax
from jax.experimental import pallas as pl
from jax.experimental.pallas import tpu as pltpu
from jax.experimental.pallas import tpu_sc as plsc

_VOCAB = 1000000
_CHUNK = 16
_NCH = 4
_ROWS = 8388608 // (_NCH * _CHUNK) // 2
_BATCH = 16384

_INFO = plsc.get_sparse_core_info()
_NC = _INFO.num_cores
_NS = _INFO.num_subcores
_NW = _NC * _NS
_NB = _BATCH // _NW
_NI = _NB * _NCH


def _body(t0_hbm, t1_hbm, r01a_hbm, out_hbm,
          r0_v, r1_v, i0_v, i1_v, p0_v, p1_v, q_v, s0, s1):
    wid = lax.axis_index("s") * _NC + lax.axis_index("c")
    base = wid * _NI

    pltpu.sync_copy(r01a_hbm.at[pl.ds(base, _NI)], r0_v)
    pltpu.sync_copy(r01a_hbm.at[pl.ds(_BATCH * _NCH + base, _NI)], r1_v)

    lane = lax.iota(jnp.int32, 16)
    col = lane & 3

    def repack(i, _):
        g0 = r0_v[pl.ds(i * 16, 16)] & (_ROWS - 1)
        g1 = r1_v[pl.ds(i * 16, 16)] & (_ROWS - 1)
        i0_v[pl.ds(i * 16, 16)] = g0 * 4 + col
        i1_v[pl.ds(i * 16, 16)] = g1 * 4 + col
        return 0

    lax.fori_loop(0, _NI // 16, repack, 0)

    cp0 = pltpu.async_copy(t0_hbm.at[i0_v], p0_v, s0)
    cp1 = pltpu.async_copy(t1_hbm.at[i1_v], p1_v, s1)
    cp0.wait()
    cp1.wait()

    def add(b, _):
        for c in range(4):
            j = b * 4 + c
            q_v[b, pl.ds(c * 16, 16)] = p0_v[j] + p1_v[j]
        return 0

    lax.fori_loop(0, _NB, add, 0)

    pltpu.sync_copy(q_v, out_hbm.at[pl.ds(wid * _NB, _NB)])


@jax.jit
def _run(x, table0, table1, h0, h1):
    rows0 = jnp.take(h0, x, axis=0)
    rows1 = jnp.take(h1, x, axis=0)
    r01f = jnp.concatenate([rows0, rows1], axis=0).reshape(2 * _BATCH * _NCH)
    t0f = table0.reshape(_ROWS * _NCH, _CHUNK)
    t1f = table1.reshape(_ROWS * _NCH, _CHUNK)

    kern = pl.kernel(
        _body,
        out_type=jax.ShapeDtypeStruct((_BATCH, _NCH * _CHUNK), jnp.float32),
        mesh=plsc.VectorSubcoreMesh(core_axis_name="c", subcore_axis_name="s"),
        compiler_params=pltpu.CompilerParams(use_tc_tiling_on_sc=False),
        scratch_types=[
            pltpu.VMEM((_NI,), jnp.int32),
            pltpu.VMEM((_NI,), jnp.int32),
            pltpu.VMEM((_NI,), jnp.int32),
            pltpu.VMEM((_NI,), jnp.int32),
            pltpu.VMEM((_NI, _CHUNK), jnp.float32),
            pltpu.VMEM((_NI, _CHUNK), jnp.float32),
            pltpu.VMEM((_NB, _NCH * _CHUNK), jnp.float32),
            pltpu.SemaphoreType.DMA,
            pltpu.SemaphoreType.DMA,
        ],
    )
    return kern(t0f, t1f, r01f)


def kernel(x, table0, table1, h0, h1):
    return _run(x, table0, table1, h0, h1)
